# baseline (speedup 1.0000x reference)
import functools

import jax
import jax.numpy as jnp
from jax import lax
from jax.experimental import pallas as pl
from jax.experimental.pallas import tpu as pltpu
from jax.experimental.pallas import tpu_sc as plsc

C = 128
TEXT_DIM = 1536
N_USERS = 4096
N_ITEMS = 4096
N_NUM = 4
N_CAT = 4
N_TXT = 1
VOCAB = 101
TOK = N_NUM + N_CAT + N_TXT

NC, NS = 2, 16
NW = NC * NS
TOTAL_CAT = 2 * N_CAT * N_USERS
ROWS_PER_W = TOTAL_CAT // NW
GCHUNK = 128
NCHUNK = ROWS_PER_W // GCHUNK
QCH = 2
NROUND = NCHUNK // QCH
NROWS = 2 * TOK * N_USERS


def _fuse_body(a_ref, b_ref, o_ref):
    o_ref[...] = a_ref[...] + b_ref[...]


def _fuse_tables(all_tab, addexp):
    return pl.pallas_call(
        _fuse_body,
        out_shape=jax.ShapeDtypeStruct((2 * N_CAT * VOCAB, C), jnp.float32),
    )(all_tab, addexp)


def _sc_fill(fused_tab, idx3):
    mesh = plsc.VectorSubcoreMesh(core_axis_name="c", subcore_axis_name="s")

    @functools.partial(
        pl.kernel,
        mesh=mesh,
        out_type=jax.ShapeDtypeStruct((NROWS, C), jnp.float32),
        scratch_types=[
            pltpu.VMEM((NCHUNK, GCHUNK), jnp.int32),
            pltpu.VMEM((QCH * GCHUNK, C), jnp.float32),
            pltpu.VMEM((QCH * GCHUNK, C), jnp.float32),
            pltpu.SemaphoreType.DMA,
            pltpu.SemaphoreType.DMA,
        ],
    )
    def k(tab_hbm, idx_hbm, out_hbm, idx_v, rows_a, rows_b, sem_a, sem_b):
        wid = lax.axis_index("s") * NC + lax.axis_index("c")
        t8 = wid // 4
        quarter = wid - 4 * t8
        side = t8 // 4
        tok = N_NUM + (t8 - 4 * side)
        rowbase = (side * TOK + tok) * N_USERS + quarter * ROWS_PER_W
        pltpu.sync_copy(idx_hbm.at[wid], idx_v)
        bufs = (rows_a, rows_b)
        sems = (sem_a, sem_b)

        def fire(q):
            buf, sem = bufs[q % 2], sems[q % 2]
            return [
                pltpu.async_copy(
                    tab_hbm.at[idx_v.at[q * QCH + j]],
                    buf.at[pl.ds(j * GCHUNK, GCHUNK)],
                    sem,
                )
                for j in range(QCH)
            ]

        inflight = [fire(0), fire(1), None, None]
        for q in range(NROUND):
            for cp in inflight[q]:
                cp.wait()
            pltpu.sync_copy(
                bufs[q % 2],
                out_hbm.at[pl.ds(rowbase + q * QCH * GCHUNK, QCH * GCHUNK)],
            )
            if q + 2 < NROUND:
                inflight[q + 2] = fire(q + 2)

    return k(fused_tab, idx3)


RT = 512
NRT = N_USERS // RT


def _text_body(x_ref, ut_ref, it_ref, tw_ref, av_ref, o_ref):
    del x_ref
    s = pl.program_id(0)
    riota = lax.broadcasted_iota(jnp.int32, (24, 1), 0)
    blk = s * TOK + 2 * N_NUM
    addv = jnp.sum(av_ref[...] * (riota == blk), axis=0, keepdims=True)
    feat = jnp.where(s == 0, ut_ref[...], it_ref[...])
    o_ref[...] = (jnp.dot(feat, tw_ref[0], preferred_element_type=jnp.float32)
                  + addv)


def _text_call(x, ut2, it2, textW, addvec):
    return pl.pallas_call(
        _text_body,
        grid=(2, NRT),
        in_specs=[
            pl.BlockSpec(memory_space=pl.ANY),
            pl.BlockSpec((RT, TEXT_DIM), lambda s, r: (jnp.where(s == 0, r, NRT - 1), 0)),
            pl.BlockSpec((RT, TEXT_DIM), lambda s, r: (jnp.where(s == 0, 0, r), 0)),
            pl.BlockSpec((1, TEXT_DIM, C), lambda s, r: (s, 0, 0)),
            pl.BlockSpec((24, C), lambda s, r: (0, 0)),
        ],
        out_specs=pl.BlockSpec(
            (RT, C), lambda s, r: (s * TOK * NRT + 2 * N_NUM * NRT + r, 0)),
        out_shape=jax.ShapeDtypeStruct((NROWS, C), jnp.float32),
        input_output_aliases={0: 0},
    )(x, ut2, it2, textW, addvec)


RN = 2048
NHALF = N_USERS // RN


def _num_body(x_ref, nfj_ref, nw_ref, nb_ref, av_ref, o_ref):
    del x_ref
    s = pl.program_id(0)
    r = pl.program_id(1)
    k = r // NHALF
    widx = s * N_NUM + k
    riota = lax.broadcasted_iota(jnp.int32, (24, 1), 0)
    addv = jnp.sum(av_ref[...] * (riota == s * TOK + k), axis=0, keepdims=True)
    c8 = lax.broadcasted_iota(jnp.int32, (1, 8), 1)
    z = jnp.sum(nfj_ref[...] * (c8 == widx), axis=1, keepdims=True)
    w8 = lax.broadcasted_iota(jnp.int32, (8, 1), 0)
    wrow = jnp.sum(nw_ref[...] * (w8 == widx), axis=0, keepdims=True)
    brow = jnp.sum(nb_ref[...] * (w8 == widx), axis=0, keepdims=True)
    zz = z * wrow + brow
    o_ref[...] = zz / (1.0 + jnp.exp(-zz)) + addv


def _num_call(x, nf_joint, numW_all, numb_all, addvec):
    nblk = N_USERS // RN
    return pl.pallas_call(
        _num_body,
        grid=(2, N_NUM * nblk),
        in_specs=[
            pl.BlockSpec(memory_space=pl.ANY),
            pl.BlockSpec((RN, 8), lambda s, r: (r % NHALF, 0)),
            pl.BlockSpec((8, C), lambda s, r: (0, 0)),
            pl.BlockSpec((8, C), lambda s, r: (0, 0)),
            pl.BlockSpec((24, C), lambda s, r: (0, 0)),
        ],
        out_specs=pl.BlockSpec(
            (RN, C), lambda s, r: (s * TOK * nblk + r, 0)),
        out_shape=jax.ShapeDtypeStruct((NROWS, C), jnp.float32),
        input_output_aliases={0: 0},
    )(x, nf_joint, numW_all, numb_all, addvec)


def kernel(users_num, users_cat, users_text, items_num, items_cat, items_text,
           table_emb, u_num_W, u_num_b, u_num_col, u_cat_tab, u_cat_col,
           u_text_W, u_text_b, u_text_col, i_num_W, i_num_b, i_num_col,
           i_cat_tab, i_cat_col, i_text_W, i_text_b, i_text_col):
    ut2 = users_text.reshape(N_USERS, TEXT_DIM)
    it2 = items_text.reshape(N_ITEMS, TEXT_DIM)
    textW = jnp.stack([u_text_W[0], i_text_W[0]])
    numW_all = jnp.concatenate([u_num_W[:, 0, :], i_num_W[:, 0, :]])
    numb_all = jnp.concatenate([u_num_b, i_num_b])
    nf_joint = jnp.concatenate([users_num, items_num], axis=1)
    te_u, te_i = table_emb[0], table_emb[1]
    addvec = jnp.concatenate([
        u_num_col + te_u,
        u_cat_col + te_u,
        u_text_col + u_text_b + te_u,
        i_num_col + te_i,
        i_cat_col + te_i,
        i_text_col + i_text_b + te_i,
        jnp.zeros((24 - 2 * TOK, C), jnp.float32),
    ])

    all_tab = jnp.concatenate([u_cat_tab, i_cat_tab]).reshape(2 * N_CAT * VOCAB, C)
    cat_add = jnp.concatenate([u_cat_col + te_u, i_cat_col + te_i])
    addexp = jnp.repeat(cat_add, VOCAB, axis=0)
    idx = jnp.concatenate([users_cat.T, items_cat.T]).astype(jnp.int32)
    idx = idx + (jnp.arange(2 * N_CAT, dtype=jnp.int32) * VOCAB)[:, None]
    idx3 = idx.reshape(NW, NCHUNK, GCHUNK)

    fused_tab = _fuse_tables(all_tab, addexp)
    x = _sc_fill(fused_tab, idx3)
    x = _text_call(x, ut2, it2, textW, addvec)
    x = _num_call(x, nf_joint, numW_all, numb_all, addvec)

    node_idxs = jnp.concatenate([
        jnp.tile(jnp.arange(N_USERS), TOK),
        jnp.tile(jnp.arange(N_USERS, N_USERS + N_ITEMS), TOK),
    ])
    table_idxs = jnp.concatenate([
        jnp.zeros(N_USERS * TOK, dtype=jnp.int32),
        jnp.ones(N_ITEMS * TOK, dtype=jnp.int32),
    ])
    col_parts = ([jnp.full((N_USERS,), c, dtype=jnp.int32) for c in range(TOK)]
                 + [jnp.full((N_ITEMS,), TOK + c, dtype=jnp.int32) for c in range(TOK)])
    col_idxs = jnp.concatenate(col_parts)
    return (x, node_idxs, col_idxs, table_idxs, N_USERS + N_ITEMS)


# speedup vs baseline: 1.1138x; 1.1138x over previous
import functools

import jax
import jax.numpy as jnp
from jax import lax
from jax.experimental import pallas as pl
from jax.experimental.pallas import tpu as pltpu
from jax.experimental.pallas import tpu_sc as plsc

C = 128
TEXT_DIM = 1536
N_USERS = 4096
N_ITEMS = 4096
N_NUM = 4
N_CAT = 4
N_TXT = 1
VOCAB = 101
TOK = N_NUM + N_CAT + N_TXT

NC, NS = 2, 16
NW = NC * NS
TOTAL_CAT = 2 * N_CAT * N_USERS
ROWS_PER_W = TOTAL_CAT // NW
GCHUNK = 128
NCHUNK = ROWS_PER_W // GCHUNK
QCH = 2
NROUND = NCHUNK // QCH
NROWS = 2 * TOK * N_USERS


def _fuse_body(a_ref, b_ref, o_ref):
    o_ref[...] = a_ref[...] + b_ref[...]


def _fuse_tables(all_tab, addexp):
    return pl.pallas_call(
        _fuse_body,
        out_shape=jax.ShapeDtypeStruct((2 * N_CAT * VOCAB, C), jnp.float32),
    )(all_tab, addexp)


def _sc_fill(fused_tab, idx3):
    mesh = plsc.VectorSubcoreMesh(core_axis_name="c", subcore_axis_name="s")

    @functools.partial(
        pl.kernel,
        mesh=mesh,
        out_type=jax.ShapeDtypeStruct((NROWS, C), jnp.float32),
        scratch_types=[
            pltpu.VMEM((NCHUNK, GCHUNK), jnp.int32),
            pltpu.VMEM((QCH * GCHUNK, C), jnp.float32),
            pltpu.VMEM((QCH * GCHUNK, C), jnp.float32),
            pltpu.SemaphoreType.DMA,
            pltpu.SemaphoreType.DMA,
        ],
    )
    def k(tab_hbm, idx_hbm, out_hbm, idx_v, rows_a, rows_b, sem_a, sem_b):
        wid = lax.axis_index("s") * NC + lax.axis_index("c")
        t8 = wid // 4
        quarter = wid - 4 * t8
        side = t8 // 4
        tok = N_NUM + (t8 - 4 * side)
        rowbase = (side * TOK + tok) * N_USERS + quarter * ROWS_PER_W
        pltpu.sync_copy(idx_hbm.at[wid], idx_v)
        bufs = (rows_a, rows_b)
        sems = (sem_a, sem_b)

        def fire(q):
            buf, sem = bufs[q % 2], sems[q % 2]
            return [
                pltpu.async_copy(
                    tab_hbm.at[idx_v.at[q * QCH + j]],
                    buf.at[pl.ds(j * GCHUNK, GCHUNK)],
                    sem,
                )
                for j in range(QCH)
            ]

        inflight = [fire(0), fire(1), None, None]
        for q in range(NROUND):
            for cp in inflight[q]:
                cp.wait()
            pltpu.sync_copy(
                bufs[q % 2],
                out_hbm.at[pl.ds(rowbase + q * QCH * GCHUNK, QCH * GCHUNK)],
            )
            if q + 2 < NROUND:
                inflight[q + 2] = fire(q + 2)

    return k(fused_tab, idx3)


RT = 1024
NRT = N_USERS // RT


def _text_body(x_ref, ut_ref, it_ref, tw_ref, av_ref, o_ref):
    del x_ref
    s = pl.program_id(0)
    riota = lax.broadcasted_iota(jnp.int32, (24, 1), 0)
    blk = s * TOK + 2 * N_NUM
    addv = jnp.sum(av_ref[...] * (riota == blk), axis=0, keepdims=True)
    feat = jnp.where(s == 0, ut_ref[...], it_ref[...])
    o_ref[...] = (jnp.dot(feat, tw_ref[0], preferred_element_type=jnp.float32)
                  + addv)


def _text_call(x, ut2, it2, textW, addvec):
    return pl.pallas_call(
        _text_body,
        grid=(2, NRT),
        in_specs=[
            pl.BlockSpec(memory_space=pl.ANY),
            pl.BlockSpec((RT, TEXT_DIM), lambda s, r: (jnp.where(s == 0, r, NRT - 1), 0)),
            pl.BlockSpec((RT, TEXT_DIM), lambda s, r: (jnp.where(s == 0, 0, r), 0)),
            pl.BlockSpec((1, TEXT_DIM, C), lambda s, r: (s, 0, 0)),
            pl.BlockSpec((24, C), lambda s, r: (0, 0)),
        ],
        out_specs=pl.BlockSpec(
            (RT, C), lambda s, r: (s * TOK * NRT + 2 * N_NUM * NRT + r, 0)),
        out_shape=jax.ShapeDtypeStruct((NROWS, C), jnp.float32),
        input_output_aliases={0: 0},
    )(x, ut2, it2, textW, addvec)


RN = 4096


def _num_body(x_ref, nfj_ref, nw_ref, nb_ref, av_ref, o_ref):
    del x_ref
    s = pl.program_id(0)
    r = pl.program_id(1)
    k = r
    widx = s * N_NUM + k
    riota = lax.broadcasted_iota(jnp.int32, (24, 1), 0)
    addv = jnp.sum(av_ref[...] * (riota == s * TOK + k), axis=0, keepdims=True)
    c8 = lax.broadcasted_iota(jnp.int32, (1, 8), 1)
    z = jnp.sum(nfj_ref[...] * (c8 == widx), axis=1, keepdims=True)
    w8 = lax.broadcasted_iota(jnp.int32, (8, 1), 0)
    wrow = jnp.sum(nw_ref[...] * (w8 == widx), axis=0, keepdims=True)
    brow = jnp.sum(nb_ref[...] * (w8 == widx), axis=0, keepdims=True)
    zz = z * wrow + brow
    o_ref[...] = zz / (1.0 + jnp.exp(-zz)) + addv


def _num_call(x, nf_joint, numW_all, numb_all, addvec):
    nblk = N_USERS // RN
    return pl.pallas_call(
        _num_body,
        grid=(2, N_NUM * nblk),
        in_specs=[
            pl.BlockSpec(memory_space=pl.ANY),
            pl.BlockSpec((RN, 8), lambda s, r: (0, 0)),
            pl.BlockSpec((8, C), lambda s, r: (0, 0)),
            pl.BlockSpec((8, C), lambda s, r: (0, 0)),
            pl.BlockSpec((24, C), lambda s, r: (0, 0)),
        ],
        out_specs=pl.BlockSpec(
            (RN, C), lambda s, r: (s * TOK * nblk + r, 0)),
        out_shape=jax.ShapeDtypeStruct((NROWS, C), jnp.float32),
        input_output_aliases={0: 0},
    )(x, nf_joint, numW_all, numb_all, addvec)


def kernel(users_num, users_cat, users_text, items_num, items_cat, items_text,
           table_emb, u_num_W, u_num_b, u_num_col, u_cat_tab, u_cat_col,
           u_text_W, u_text_b, u_text_col, i_num_W, i_num_b, i_num_col,
           i_cat_tab, i_cat_col, i_text_W, i_text_b, i_text_col):
    ut2 = users_text.reshape(N_USERS, TEXT_DIM)
    it2 = items_text.reshape(N_ITEMS, TEXT_DIM)
    textW = jnp.stack([u_text_W[0], i_text_W[0]])
    numW_all = jnp.concatenate([u_num_W[:, 0, :], i_num_W[:, 0, :]])
    numb_all = jnp.concatenate([u_num_b, i_num_b])
    nf_joint = jnp.concatenate([users_num, items_num], axis=1)
    te_u, te_i = table_emb[0], table_emb[1]
    addvec = jnp.concatenate([
        u_num_col + te_u,
        u_cat_col + te_u,
        u_text_col + u_text_b + te_u,
        i_num_col + te_i,
        i_cat_col + te_i,
        i_text_col + i_text_b + te_i,
        jnp.zeros((24 - 2 * TOK, C), jnp.float32),
    ])

    all_tab = jnp.concatenate([u_cat_tab, i_cat_tab]).reshape(2 * N_CAT * VOCAB, C)
    cat_add = jnp.concatenate([u_cat_col + te_u, i_cat_col + te_i])
    addexp = jnp.repeat(cat_add, VOCAB, axis=0)
    idx = jnp.concatenate([users_cat.T, items_cat.T]).astype(jnp.int32)
    idx = idx + (jnp.arange(2 * N_CAT, dtype=jnp.int32) * VOCAB)[:, None]
    idx3 = idx.reshape(NW, NCHUNK, GCHUNK)

    fused_tab = _fuse_tables(all_tab, addexp)
    x = _sc_fill(fused_tab, idx3)
    x = _text_call(x, ut2, it2, textW, addvec)
    x = _num_call(x, nf_joint, numW_all, numb_all, addvec)

    node_idxs = jnp.concatenate([
        jnp.tile(jnp.arange(N_USERS), TOK),
        jnp.tile(jnp.arange(N_USERS, N_USERS + N_ITEMS), TOK),
    ])
    table_idxs = jnp.concatenate([
        jnp.zeros(N_USERS * TOK, dtype=jnp.int32),
        jnp.ones(N_ITEMS * TOK, dtype=jnp.int32),
    ])
    col_parts = ([jnp.full((N_USERS,), c, dtype=jnp.int32) for c in range(TOK)]
                 + [jnp.full((N_ITEMS,), TOK + c, dtype=jnp.int32) for c in range(TOK)])
    col_idxs = jnp.concatenate(col_parts)
    return (x, node_idxs, col_idxs, table_idxs, N_USERS + N_ITEMS)


# speedup vs baseline: 1.1151x; 1.0012x over previous
import functools

import jax
import jax.numpy as jnp
from jax import lax
from jax.experimental import pallas as pl
from jax.experimental.pallas import tpu as pltpu
from jax.experimental.pallas import tpu_sc as plsc

C = 128
TEXT_DIM = 1536
N_USERS = 4096
N_ITEMS = 4096
N_NUM = 4
N_CAT = 4
N_TXT = 1
VOCAB = 101
TOK = N_NUM + N_CAT + N_TXT

NC, NS = 2, 16
NW = NC * NS
TOTAL_CAT = 2 * N_CAT * N_USERS
ROWS_PER_W = TOTAL_CAT // NW
GCHUNK = 128
NROUND = ROWS_PER_W // GCHUNK
NBUF = 4
NROWS = 2 * TOK * N_USERS


def _fuse_body(ut_ref, it_ref, uc_ref, ic_ref, te_ref, o_ref):
    for t in range(N_CAT):
        o_ref[t] = (ut_ref[t] + uc_ref[pl.ds(t, 1), :]
                    + te_ref[pl.ds(0, 1), :])
        o_ref[N_CAT + t] = (it_ref[t] + ic_ref[pl.ds(t, 1), :]
                            + te_ref[pl.ds(1, 1), :])


def _fuse_tables(u_cat_tab, i_cat_tab, u_cat_col, i_cat_col, table_emb):
    return pl.pallas_call(
        _fuse_body,
        out_shape=jax.ShapeDtypeStruct((2 * N_CAT, VOCAB, C), jnp.float32),
    )(u_cat_tab, i_cat_tab, u_cat_col, i_cat_col, table_emb)


def _sc_fill(fused_tab, idx3):
    mesh = plsc.VectorSubcoreMesh(core_axis_name="c", subcore_axis_name="s")

    @functools.partial(
        pl.kernel,
        mesh=mesh,
        out_type=jax.ShapeDtypeStruct((NROWS, C), jnp.float32),
        scratch_types=[
            pltpu.VMEM((NROUND, GCHUNK), jnp.int32),
            pltpu.VMEM((NBUF, GCHUNK, C), jnp.float32),
            pltpu.SemaphoreType.DMA((NBUF,)),
            pltpu.SemaphoreType.DMA((NBUF,)),
        ],
    )
    def k(tab_hbm, idx_hbm, out_hbm, idx_v, rows_v, gsem, osem):
        wid = lax.axis_index("s") * NC + lax.axis_index("c")
        t8 = wid // 4
        quarter = wid - 4 * t8
        side = t8 // 4
        tok = N_NUM + (t8 - 4 * side)
        rowbase = (side * TOK + tok) * N_USERS + quarter * ROWS_PER_W
        pltpu.sync_copy(idx_hbm.at[wid], idx_v)

        def fire(q):
            return pltpu.async_copy(
                tab_hbm.at[idx_v.at[q]], rows_v.at[q % NBUF],
                gsem.at[q % NBUF])

        gathers = [None] * NROUND
        outs = [None] * NROUND
        for q in range(NBUF):
            gathers[q] = fire(q)
        for q in range(NROUND):
            gathers[q].wait()
            outs[q] = pltpu.async_copy(
                rows_v.at[q % NBUF],
                out_hbm.at[pl.ds(rowbase + q * GCHUNK, GCHUNK)],
                osem.at[q % NBUF])
            if q + NBUF < NROUND:
                outs[q].wait()
                gathers[q + NBUF] = fire(q + NBUF)
        for q in range(NROUND - NBUF, NROUND):
            outs[q].wait()

    return k(fused_tab, idx3)


RT = 1024
NRT = N_USERS // RT


def _text_body(x_ref, ut_ref, it_ref, tw_ref, utc_ref, utb_ref, itc_ref,
               itb_ref, te_ref, o_ref):
    del x_ref
    s = pl.program_id(0)
    terow = jnp.where(s == 0, te_ref[pl.ds(0, 1), :], te_ref[pl.ds(1, 1), :])
    addv = jnp.where(s == 0, utc_ref[...] + utb_ref[...],
                     itc_ref[...] + itb_ref[...]) + terow
    feat = jnp.where(s == 0, ut_ref[...], it_ref[...])
    o_ref[...] = (jnp.dot(feat, tw_ref[0], preferred_element_type=jnp.float32)
                  + addv)


def _text_call(x, ut2, it2, textW, u_text_col, u_text_b, i_text_col, i_text_b,
               table_emb):
    return pl.pallas_call(
        _text_body,
        grid=(2, NRT),
        in_specs=[
            pl.BlockSpec(memory_space=pl.ANY),
            pl.BlockSpec((RT, TEXT_DIM), lambda s, r: (jnp.where(s == 0, r, NRT - 1), 0)),
            pl.BlockSpec((RT, TEXT_DIM), lambda s, r: (jnp.where(s == 0, 0, r), 0)),
            pl.BlockSpec((1, TEXT_DIM, C), lambda s, r: (s, 0, 0)),
            pl.BlockSpec((1, C), lambda s, r: (0, 0)),
            pl.BlockSpec((1, C), lambda s, r: (0, 0)),
            pl.BlockSpec((1, C), lambda s, r: (0, 0)),
            pl.BlockSpec((1, C), lambda s, r: (0, 0)),
            pl.BlockSpec((2, C), lambda s, r: (0, 0)),
        ],
        out_specs=pl.BlockSpec(
            (RT, C), lambda s, r: (s * TOK * NRT + 2 * N_NUM * NRT + r, 0)),
        out_shape=jax.ShapeDtypeStruct((NROWS, C), jnp.float32),
        input_output_aliases={0: 0},
    )(x, ut2, it2, textW, u_text_col, u_text_b, i_text_col, i_text_b,
      table_emb)


def _num_body(x_ref, un_ref, in_ref, uw_ref, iw_ref, ub_ref, ib_ref, uc_ref,
              ic_ref, te_ref, o_ref):
    del x_ref
    s = pl.program_id(0)
    k = pl.program_id(1)
    nf = jnp.where(s == 0, un_ref[...], in_ref[...])
    c4 = lax.broadcasted_iota(jnp.int32, (1, N_NUM), 1)
    z = jnp.sum(nf * (c4 == k), axis=1, keepdims=True)
    w4 = lax.broadcasted_iota(jnp.int32, (N_NUM, 1), 0)
    wsel = jnp.where(s == 0, uw_ref[...][:, 0, :], iw_ref[...][:, 0, :])
    wrow = jnp.sum(wsel * (w4 == k), axis=0, keepdims=True)
    bsel = jnp.where(s == 0, ub_ref[...], ib_ref[...])
    brow = jnp.sum(bsel * (w4 == k), axis=0, keepdims=True)
    csel = jnp.where(s == 0, uc_ref[...], ic_ref[...])
    crow = jnp.sum(csel * (w4 == k), axis=0, keepdims=True)
    terow = jnp.where(s == 0, te_ref[pl.ds(0, 1), :], te_ref[pl.ds(1, 1), :])
    zz = z * wrow + brow
    o_ref[...] = zz / (1.0 + jnp.exp(-zz)) + crow + terow


def _num_call(x, users_num, items_num, u_num_W, i_num_W, u_num_b, i_num_b,
              u_num_col, i_num_col, table_emb):
    full = lambda s, r: (0, 0)
    return pl.pallas_call(
        _num_body,
        grid=(2, N_NUM),
        in_specs=[
            pl.BlockSpec(memory_space=pl.ANY),
            pl.BlockSpec((N_USERS, N_NUM), full),
            pl.BlockSpec((N_ITEMS, N_NUM), full),
            pl.BlockSpec((N_NUM, 1, C), lambda s, r: (0, 0, 0)),
            pl.BlockSpec((N_NUM, 1, C), lambda s, r: (0, 0, 0)),
            pl.BlockSpec((N_NUM, C), full),
            pl.BlockSpec((N_NUM, C), full),
            pl.BlockSpec((N_NUM, C), full),
            pl.BlockSpec((N_NUM, C), full),
            pl.BlockSpec((2, C), full),
        ],
        out_specs=pl.BlockSpec(
            (N_USERS, C), lambda s, r: (s * TOK + r, 0)),
        out_shape=jax.ShapeDtypeStruct((NROWS, C), jnp.float32),
        input_output_aliases={0: 0},
    )(x, users_num, items_num, u_num_W, i_num_W, u_num_b, i_num_b,
      u_num_col, i_num_col, table_emb)


def kernel(users_num, users_cat, users_text, items_num, items_cat, items_text,
           table_emb, u_num_W, u_num_b, u_num_col, u_cat_tab, u_cat_col,
           u_text_W, u_text_b, u_text_col, i_num_W, i_num_b, i_num_col,
           i_cat_tab, i_cat_col, i_text_W, i_text_b, i_text_col):
    ut2 = users_text.reshape(N_USERS, TEXT_DIM)
    it2 = items_text.reshape(N_ITEMS, TEXT_DIM)
    textW = jnp.stack([u_text_W[0], i_text_W[0]])
    idx = jnp.concatenate([users_cat.T, items_cat.T]).astype(jnp.int32)
    idx = idx + (jnp.arange(2 * N_CAT, dtype=jnp.int32) * VOCAB)[:, None]
    idx3 = idx.reshape(NW, NROUND, GCHUNK)

    fused_tab = _fuse_tables(u_cat_tab, i_cat_tab, u_cat_col, i_cat_col,
                             table_emb)
    x = _sc_fill(fused_tab.reshape(2 * N_CAT * VOCAB, C), idx3)
    x = _text_call(x, ut2, it2, textW, u_text_col, u_text_b, i_text_col,
                   i_text_b, table_emb)
    x = _num_call(x, users_num, items_num, u_num_W, i_num_W, u_num_b,
                  i_num_b, u_num_col, i_num_col, table_emb)

    node_idxs = jnp.concatenate([
        jnp.tile(jnp.arange(N_USERS), TOK),
        jnp.tile(jnp.arange(N_USERS, N_USERS + N_ITEMS), TOK),
    ])
    table_idxs = jnp.concatenate([
        jnp.zeros(N_USERS * TOK, dtype=jnp.int32),
        jnp.ones(N_ITEMS * TOK, dtype=jnp.int32),
    ])
    col_parts = ([jnp.full((N_USERS,), c, dtype=jnp.int32) for c in range(TOK)]
                 + [jnp.full((N_ITEMS,), TOK + c, dtype=jnp.int32) for c in range(TOK)])
    col_idxs = jnp.concatenate(col_parts)
    return (x, node_idxs, col_idxs, table_idxs, N_USERS + N_ITEMS)
